# baseline (speedup 1.0000x reference)
import functools

import jax
import jax.numpy as jnp
import numpy as np
from jax import lax
from jax.experimental import pallas as pl
from jax.experimental.pallas import tpu as pltpu

_B, _S, _H = 4, 8192, 2048
_AD, _R, _K = 32, 16, 8
_TEMP = 0.1

_RC = 1024
_CPB = _S // _RC
_N_STEPS = (_B * _S) // _RC


def _tail(pooled, W1, b1, gamma, beta, W2, b2, mask_logits, keep, gumbel,
          training):
    x = jnp.dot(pooled, W1, preferred_element_type=jnp.float32) + b1
    mu = jnp.mean(x, axis=-1, keepdims=True)
    var = jnp.mean((x - mu) ** 2, axis=-1, keepdims=True)
    x = (x - mu) / jnp.sqrt(var + 1e-5) * gamma + beta
    x = 0.5 * x * (1.0 + lax.erf(x / jnp.sqrt(2.0).astype(jnp.float32)))
    x_dropped = jnp.where(keep > 0.5, x / 0.9, 0.0)
    is_training = training != 0
    x = jnp.where(is_training, x_dropped, x)
    logits = (jnp.dot(x, W2, preferred_element_type=jnp.float32) + b2
              + mask_logits)

    def khot(v):
        col = lax.broadcasted_iota(jnp.int32, v.shape, 1)
        cnt = jnp.zeros(v.shape, jnp.int32)
        for k in range(_R):
            vk = v[:, k][:, None]
            beats = (vk > v) | ((vk == v) & (k < col))
            cnt = cnt + beats.astype(jnp.int32)
        return (cnt < _K).astype(jnp.float32)

    noisy = (logits + gumbel) / _TEMP
    hard = khot(noisy)
    z = logits / _TEMP
    z = z - jnp.max(z, axis=-1, keepdims=True)
    ez = jnp.exp(z)
    soft = ez / jnp.sum(ez, axis=-1, keepdims=True)
    mask_train = hard + soft - soft
    mask_eval = khot(logits)
    return jnp.where(is_training, mask_train, mask_eval)


def _fused_kernel(hs_ref, W1_ref, b1_ref, gamma_ref, beta_ref, W2_ref,
                  b2_ref, ml_ref, keep_ref, gumbel_ref, train_ref,
                  out_ref, acc_ref):
    c = pl.program_id(0)
    bb = c // _CPB

    @pl.when(c == 0)
    def _():
        acc_ref[...] = jnp.zeros_like(acc_ref)

    acc_ref[pl.ds(bb * 8, 8), :] += jnp.sum(hs_ref[...], axis=0)

    @pl.when(c == _N_STEPS - 1)
    def _():
        pooled = jnp.sum(acc_ref[...].reshape(_B, 8, _H), axis=1) * (1.0 / _S)
        out_ref[...] = _tail(
            pooled, W1_ref[...], b1_ref[...], gamma_ref[...], beta_ref[...],
            W2_ref[...], b2_ref[...], ml_ref[...], keep_ref[...],
            gumbel_ref[...], train_ref[0, 0])


_RNG_CONSTS = None


def _rng_consts():
    global _RNG_CONSTS
    if _RNG_CONSTS is None:
        try:
            cpu = jax.local_devices(backend="cpu")[0]
            with jax.default_device(cpu):
                keep = np.asarray(
                    jax.random.bernoulli(jax.random.key(42), 0.9,
                                         (_B, _AD))).astype(np.float32)
                u = np.asarray(
                    jax.random.uniform(jax.random.key(7), (_B, _R),
                                       dtype=jnp.float32))
            gumbel = (-np.log(-np.log(u + np.float32(1e-8))
                              + np.float32(1e-8))).astype(np.float32)
            _RNG_CONSTS = (keep, gumbel)
        except Exception:
            keep = jax.random.bernoulli(jax.random.key(42), 0.9,
                                        (_B, _AD)).astype(jnp.float32)
            u = jax.random.uniform(jax.random.key(7), (_B, _R),
                                   dtype=jnp.float32)
            gumbel = -jnp.log(-jnp.log(u + 1e-8) + 1e-8)
            return keep, gumbel
    return _RNG_CONSTS


def kernel(hidden_states, W1, b1, gamma, beta, W2, b2, mask_logits,
           training):
    keep_c, gumbel_c = _rng_consts()
    keep = jnp.asarray(keep_c, jnp.float32)
    gumbel = jnp.asarray(gumbel_c, jnp.float32)
    train_arr = jnp.asarray(training, jnp.float32).reshape(1, 1)

    hs3 = hidden_states.reshape((_B * _S) // 8, 8, _H)
    tiny = lambda r, c: pl.BlockSpec((r, c), lambda i: (0, 0))
    return pl.pallas_call(
        _fused_kernel,
        grid=(_N_STEPS,),
        in_specs=[
            pl.BlockSpec((_RC // 8, 8, _H), lambda i: (i, 0, 0)),
            tiny(_H, _AD),
            tiny(1, _AD),
            tiny(1, _AD),
            tiny(1, _AD),
            tiny(_AD, _R),
            tiny(1, _R),
            tiny(1, _R),
            tiny(_B, _AD),
            tiny(_B, _R),
            tiny(1, 1),
        ],
        out_specs=pl.BlockSpec((_B, _R), lambda i: (0, 0)),
        out_shape=jax.ShapeDtypeStruct((_B, _R), jnp.float32),
        scratch_shapes=[pltpu.VMEM((_B * 8, _H), jnp.float32)],
    )(hs3, W1, b1.reshape(1, _AD), gamma.reshape(1, _AD),
      beta.reshape(1, _AD), W2, b2.reshape(1, _R),
      mask_logits.reshape(1, _R), keep, gumbel, train_arr)


# speedup vs baseline: 1.0664x; 1.0664x over previous
import functools

import jax
import jax.numpy as jnp
import numpy as np
from jax import lax
from jax.experimental import pallas as pl
from jax.experimental.pallas import tpu as pltpu

_B, _S, _H = 4, 8192, 2048
_AD, _R, _K = 32, 16, 8
_TEMP = 0.1

_RC = 1024
_CPB = _S // _RC
_N_STEPS = (_B * _S) // _RC

_KEEP = np.array([
    1, 1, 1, 1, 1, 1, 1, 1, 1, 1, 1, 1, 0, 1, 1, 1, 1, 1, 1, 1, 1, 1, 1, 1,
    1, 1, 1, 1, 1, 0, 1, 1, 1, 1, 1, 1, 1, 1, 1, 1, 1, 1, 1, 1, 1, 1, 1, 1,
    1, 1, 1, 1, 1, 0, 1, 1, 1, 1, 1, 1, 1, 1, 1, 1, 1, 1, 0, 1, 1, 1, 1, 1,
    1, 1, 1, 0, 1, 1, 1, 1, 1, 0, 1, 1, 1, 1, 1, 1, 0, 1, 1, 1, 1, 1, 1, 1,
    1, 1, 1, 1, 1, 1, 1, 1, 1, 1, 1, 1, 1, 1, 1, 1, 1, 1, 1, 1, 1, 1, 1, 1,
    1, 1, 1, 1, 0, 1, 1, 1], np.float32).reshape(_B, _AD)
_U_BITS = np.array([
    1059885352, 1064927358, 1050349136, 1055084168, 1060838242, 1059161242,
    1055238244, 1053994408, 1032773744, 1035453824, 1052642588, 1062689852,
    1060341856, 1025859872, 1064174752, 1056459868, 1053590928, 1061458198,
    1052466412, 1064925040, 1064264504, 1053505852, 1051941012, 1059972726,
    1061245072, 1038974496, 1061770010, 1052207096, 1057380956, 1050048952,
    1064633894, 1022733184, 1063964812, 1049481120, 1021247104, 1064524462,
    1034229040, 1057307632, 1058407700, 1052001344, 1042932960, 1062645412,
    1044291112, 1055475592, 1057955984, 1062123842, 1051846184, 1060536782,
    1053466628, 1049533380, 1054298048, 1056642184, 1044688472, 1060244302,
    1058937014, 1058888434, 1048937460, 1060527170, 1057281386, 1054766964,
    1049556740, 1064787876, 1026640864, 1053030028], np.uint32)
_U = _U_BITS.view(np.float32).reshape(_B, _R)
_GUMBEL = (-np.log(-np.log(_U + np.float32(1e-8))
                   + np.float32(1e-8))).astype(np.float32)


def _tail(pooled, W1t, b1, gamma, beta, W2t, b2, mask_logits, keep, gumbel,
          training):
    cdn = (((1,), (1,)), ((), ()))
    x = lax.dot_general(pooled, W1t, dimension_numbers=cdn,
                        preferred_element_type=jnp.float32) + b1
    mu = jnp.mean(x, axis=-1, keepdims=True)
    var = jnp.mean((x - mu) ** 2, axis=-1, keepdims=True)
    x = (x - mu) / jnp.sqrt(var + 1e-5) * gamma + beta
    x = 0.5 * x * (1.0 + lax.erf(x / jnp.sqrt(2.0).astype(jnp.float32)))
    x_dropped = jnp.where(keep > 0.5, x / 0.9, 0.0)
    is_training = training != 0
    x = jnp.where(is_training, x_dropped, x)
    logits = lax.dot_general(x, W2t, dimension_numbers=cdn,
                             preferred_element_type=jnp.float32)
    logits = logits + b2 + mask_logits

    def khot(v):
        col = lax.broadcasted_iota(jnp.int32, v.shape, 1)
        cnt = jnp.zeros(v.shape, jnp.int32)
        for k in range(_R):
            vk = v[:, k][:, None]
            beats = (vk > v) | ((vk == v) & (k < col))
            cnt = cnt + beats.astype(jnp.int32)
        return (cnt < _K).astype(jnp.float32)

    noisy = (logits + gumbel) / _TEMP
    hard = khot(noisy)
    z = logits / _TEMP
    z = z - jnp.max(z, axis=-1, keepdims=True)
    ez = jnp.exp(z)
    soft = ez / jnp.sum(ez, axis=-1, keepdims=True)
    mask_train = hard + soft - soft
    mask_eval = khot(logits)
    return jnp.where(is_training, mask_train, mask_eval)


def _fused_kernel(hs_ref, W1t_ref, b1_ref, gamma_ref, beta_ref, W2t_ref,
                  b2_ref, ml_ref, keep_ref, gumbel_ref, train_ref,
                  out_ref, acc_ref):
    c = pl.program_id(0)
    bb = c // _CPB

    @pl.when(c == 0)
    def _():
        acc_ref[...] = jnp.zeros_like(acc_ref)

    acc_ref[pl.ds(bb * 8, 8), :] += jnp.sum(hs_ref[...], axis=0)

    @pl.when(c == _N_STEPS - 1)
    def _():
        pooled = jnp.sum(acc_ref[...].reshape(_B, 8, _H), axis=1) * (1.0 / _S)
        out_ref[...] = _tail(
            pooled, W1t_ref[...], b1_ref[...], gamma_ref[...], beta_ref[...],
            W2t_ref[...], b2_ref[...], ml_ref[...], keep_ref[...],
            gumbel_ref[...], train_ref[0, 0])


def kernel(hidden_states, W1, b1, gamma, beta, W2, b2, mask_logits,
           training):
    keep = jnp.asarray(_KEEP)
    gumbel = jnp.asarray(_GUMBEL)
    train_arr = jnp.asarray(training, jnp.float32).reshape(1, 1)

    hs3 = hidden_states.reshape((_B * _S) // 8, 8, _H)
    tiny = lambda r, c: pl.BlockSpec((r, c), lambda i: (0, 0))
    return pl.pallas_call(
        _fused_kernel,
        grid=(_N_STEPS,),
        in_specs=[
            pl.BlockSpec((_RC // 8, 8, _H), lambda i: (i, 0, 0)),
            tiny(_AD, _H),
            tiny(1, _AD),
            tiny(1, _AD),
            tiny(1, _AD),
            tiny(_R, _AD),
            tiny(1, _R),
            tiny(1, _R),
            tiny(_B, _AD),
            tiny(_B, _R),
            tiny(1, 1),
        ],
        out_specs=pl.BlockSpec((_B, _R), lambda i: (0, 0)),
        out_shape=jax.ShapeDtypeStruct((_B, _R), jnp.float32),
        scratch_shapes=[pltpu.VMEM((_B * 8, _H), jnp.float32)],
    )(hs3, W1.T, b1.reshape(1, _AD), gamma.reshape(1, _AD),
      beta.reshape(1, _AD), W2.T, b2.reshape(1, _R),
      mask_logits.reshape(1, _R), keep, gumbel, train_arr)


# speedup vs baseline: 1.0673x; 1.0009x over previous
import jax
import jax.numpy as jnp
import numpy as np
from jax import lax
from jax.experimental import pallas as pl
from jax.experimental.pallas import tpu as pltpu

_B, _S, _H = 4, 8192, 2048
_AD, _R, _K = 32, 16, 8
_TEMP = 0.1

_RC = 1024
_CPB = _S // _RC
_N_STEPS = (_B * _S) // _RC

_KEEP = np.array([
    1, 1, 1, 1, 1, 1, 1, 1, 1, 1, 1, 1, 0, 1, 1, 1, 1, 1, 1, 1, 1, 1, 1, 1,
    1, 1, 1, 1, 1, 0, 1, 1, 1, 1, 1, 1, 1, 1, 1, 1, 1, 1, 1, 1, 1, 1, 1, 1,
    1, 1, 1, 1, 1, 0, 1, 1, 1, 1, 1, 1, 1, 1, 1, 1, 1, 1, 0, 1, 1, 1, 1, 1,
    1, 1, 1, 0, 1, 1, 1, 1, 1, 0, 1, 1, 1, 1, 1, 1, 0, 1, 1, 1, 1, 1, 1, 1,
    1, 1, 1, 1, 1, 1, 1, 1, 1, 1, 1, 1, 1, 1, 1, 1, 1, 1, 1, 1, 1, 1, 1, 1,
    1, 1, 1, 1, 0, 1, 1, 1], np.float32).reshape(_B, _AD)
_U_BITS = np.array([
    1059885352, 1064927358, 1050349136, 1055084168, 1060838242, 1059161242,
    1055238244, 1053994408, 1032773744, 1035453824, 1052642588, 1062689852,
    1060341856, 1025859872, 1064174752, 1056459868, 1053590928, 1061458198,
    1052466412, 1064925040, 1064264504, 1053505852, 1051941012, 1059972726,
    1061245072, 1038974496, 1061770010, 1052207096, 1057380956, 1050048952,
    1064633894, 1022733184, 1063964812, 1049481120, 1021247104, 1064524462,
    1034229040, 1057307632, 1058407700, 1052001344, 1042932960, 1062645412,
    1044291112, 1055475592, 1057955984, 1062123842, 1051846184, 1060536782,
    1053466628, 1049533380, 1054298048, 1056642184, 1044688472, 1060244302,
    1058937014, 1058888434, 1048937460, 1060527170, 1057281386, 1054766964,
    1049556740, 1064787876, 1026640864, 1053030028], np.uint32)
_U = _U_BITS.view(np.float32).reshape(_B, _R)
_GUMBEL = (-np.log(-np.log(_U + np.float32(1e-8))
                   + np.float32(1e-8))).astype(np.float32)


def _tail(pooled, W1t, b1, gamma, beta, W2t, b2, mask_logits, keep, gumbel,
          training):
    cdn = (((1,), (1,)), ((), ()))
    x = lax.dot_general(pooled, W1t, dimension_numbers=cdn,
                        preferred_element_type=jnp.float32) + b1
    mu = jnp.mean(x, axis=-1, keepdims=True)
    var = jnp.mean((x - mu) ** 2, axis=-1, keepdims=True)
    x = (x - mu) / jnp.sqrt(var + 1e-5) * gamma + beta
    x = 0.5 * x * (1.0 + lax.erf(x / jnp.sqrt(2.0).astype(jnp.float32)))
    x_dropped = jnp.where(keep > 0.5, x / 0.9, 0.0)
    is_training = training != 0
    x = jnp.where(is_training, x_dropped, x)
    logits = lax.dot_general(x, W2t, dimension_numbers=cdn,
                             preferred_element_type=jnp.float32)
    logits = logits + b2 + mask_logits

    def khot(v):
        col = lax.broadcasted_iota(jnp.int32, v.shape, 1)
        cnt = jnp.zeros(v.shape, jnp.int32)
        for k in range(_R):
            vk = v[:, k][:, None]
            beats = (vk > v) | ((vk == v) & (k < col))
            cnt = cnt + beats.astype(jnp.int32)
        return (cnt < _K).astype(jnp.float32)

    noisy = (logits + gumbel) / _TEMP
    hard = khot(noisy)
    z = logits / _TEMP
    z = z - jnp.max(z, axis=-1, keepdims=True)
    ez = jnp.exp(z)
    soft = ez / jnp.sum(ez, axis=-1, keepdims=True)
    mask_train = hard + soft - soft
    mask_eval = khot(logits)
    return jnp.where(is_training, mask_train, mask_eval)


def _fused_kernel(hs_ref, W1t_ref, b1_ref, gamma_ref, beta_ref, W2t_ref,
                  b2_ref, ml_ref, keep_ref, gumbel_ref, train_ref,
                  out_ref, acc_ref):
    c = pl.program_id(0)
    bb = c // _CPB

    @pl.when(c == 0)
    def _():
        acc_ref[...] = jnp.zeros_like(acc_ref)

    acc_ref[pl.ds(bb * 8, 8), :] += jnp.sum(hs_ref[...], axis=0)

    @pl.when(c == _N_STEPS - 1)
    def _():
        pooled = jnp.sum(acc_ref[...].reshape(_B, 8, _H), axis=1) * (1.0 / _S)
        out_ref[...] = _tail(
            pooled, W1t_ref[...], b1_ref[...], gamma_ref[...], beta_ref[...],
            W2t_ref[...], b2_ref[...], ml_ref[...], keep_ref[...],
            gumbel_ref[...], train_ref[0, 0])


def kernel(hidden_states, W1, b1, gamma, beta, W2, b2, mask_logits,
           training):
    keep = jnp.asarray(_KEEP)
    gumbel = jnp.asarray(_GUMBEL)
    train_arr = jnp.asarray(training, jnp.float32).reshape(1, 1)

    hs3 = hidden_states.reshape((_B * _S) // 8, 8, _H)
    tiny = lambda r, c: pl.BlockSpec((r, c), lambda i: (0, 0))
    return pl.pallas_call(
        _fused_kernel,
        grid=(_N_STEPS,),
        in_specs=[
            pl.BlockSpec((_RC // 8, 8, _H), lambda i: (i, 0, 0)),
            tiny(_AD, _H),
            tiny(1, _AD),
            tiny(1, _AD),
            tiny(1, _AD),
            tiny(_R, _AD),
            tiny(1, _R),
            tiny(1, _R),
            tiny(_B, _AD),
            tiny(_B, _R),
            tiny(1, 1),
        ],
        out_specs=pl.BlockSpec((_B, _R), lambda i: (0, 0)),
        out_shape=jax.ShapeDtypeStruct((_B, _R), jnp.float32),
        scratch_shapes=[pltpu.VMEM((_B * 8, _H), jnp.float32)],
    )(hs3, W1.T, b1.reshape(1, _AD), gamma.reshape(1, _AD),
      beta.reshape(1, _AD), W2.T, b2.reshape(1, _R),
      mask_logits.reshape(1, _R), keep, gumbel, train_arr)
